# baseline (speedup 1.0000x reference)
import functools

import jax
import jax.numpy as jnp
from jax import lax
from jax.experimental import pallas as pl
from jax.experimental.pallas import tpu as pltpu
from jax.experimental.pallas import tpu_sc as plsc

BATCH = 4
SEQ_LEN = 2048
D_MODEL = 1024

NUM_CORES = 2
NUM_SUBCORES = 16
ROWS_PER_CHUNK = 16
NBUF = 4
VECS = ROWS_PER_CHUNK * D_MODEL // 16
LANES_PER_ROW = D_MODEL // 16

CHUNKS_C0 = 5
CHUNKS_C1 = 3
ROWS_C0 = CHUNKS_C0 * ROWS_PER_CHUNK
ROWS_C1 = CHUNKS_C1 * ROWS_PER_CHUNK
SPLIT = NUM_SUBCORES * ROWS_C0

_mesh = plsc.VectorSubcoreMesh(core_axis_name="c", subcore_axis_name="s")

_scratch = (
    [pltpu.VMEM((ROWS_PER_CHUNK, D_MODEL), jnp.float32) for _ in range(2)]
    + [pltpu.VMEM((ROWS_PER_CHUNK, D_MODEL), jnp.float32) for _ in range(NBUF)]
    + [pltpu.SemaphoreType.DMA for _ in range(2 + 2 * NBUF)]
)


@functools.partial(
    pl.kernel,
    mesh=_mesh,
    out_type=jax.ShapeDtypeStruct((BATCH, SEQ_LEN, D_MODEL), jnp.float32),
    scratch_types=_scratch,
)
def _pos_enc_sc(x_hbm, emb_hbm, out_hbm, *bufs):
    ebufs = bufs[0:2]
    xbufs = bufs[2 : 2 + NBUF]
    esems = bufs[2 + NBUF : 4 + NBUF]
    isems = bufs[4 + NBUF : 4 + 2 * NBUF]
    osems = bufs[4 + 2 * NBUF : 4 + 3 * NBUF]

    cid = lax.axis_index("c")
    sid = lax.axis_index("s")

    def pipeline(seq_base, chunks):
        jobs = chunks * BATCH

        def seq0(c):
            return seq_base + c * ROWS_PER_CHUNK

        def issue_in(j):
            c, b = divmod(j, BATCH)
            return pltpu.async_copy(
                x_hbm.at[b, pl.ds(seq0(c), ROWS_PER_CHUNK)],
                xbufs[j % NBUF],
                isems[j % NBUF],
            )

        def issue_out(j):
            c, b = divmod(j, BATCH)
            return pltpu.async_copy(
                xbufs[j % NBUF],
                out_hbm.at[b, pl.ds(seq0(c), ROWS_PER_CHUNK)],
                osems[j % NBUF],
            )

        def issue_emb(c):
            return pltpu.async_copy(
                emb_hbm.at[pl.ds(seq0(c), ROWS_PER_CHUNK)], ebufs[c % 2], esems[c % 2]
            )

        ecpys = [None] * chunks
        incpy = [None] * jobs
        outcpy = [None] * jobs

        ecpys[0] = issue_emb(0)
        if chunks > 1:
            ecpys[1] = issue_emb(1)
        for j in range(NBUF - 1):
            incpy[j] = issue_in(j)

        for j in range(jobs):
            c, b = divmod(j, BATCH)
            nj = j + NBUF - 1
            if nj < jobs:
                if nj - NBUF >= 0:
                    outcpy[nj - NBUF].wait()
                incpy[nj] = issue_in(nj)
            if b == 0:
                ecpys[c].wait()
            incpy[j].wait()

            ebuf = ebufs[c % 2]

            @plsc.parallel_loop(0, VECS, step=1, unroll=8)
            def _add(i, buf=xbufs[j % NBUF], ebuf=ebuf):
                r = i // LANES_PER_ROW
                col = (i % LANES_PER_ROW) * 16
                sl = pl.ds(col, 16)
                plsc.addupdate(buf.at[r, sl], ebuf[r, sl])

            if b == BATCH - 1 and c + 2 < chunks:
                ecpys[c + 2] = issue_emb(c + 2)

            outcpy[j] = issue_out(j)

        for j in range(max(jobs - NBUF, 0), jobs):
            outcpy[j].wait()

    @pl.when(cid == 0)
    def _():
        pipeline(sid * ROWS_C0, CHUNKS_C0)

    @pl.when(cid == 1)
    def _():
        pipeline(SPLIT + sid * ROWS_C1, CHUNKS_C1)


def kernel(x, emb):
    return _pos_enc_sc(x, emb)


# speedup vs baseline: 1.0155x; 1.0155x over previous
import functools

import jax
import jax.numpy as jnp
from jax import lax
from jax.experimental import pallas as pl
from jax.experimental.pallas import tpu as pltpu
from jax.experimental.pallas import tpu_sc as plsc

BATCH = 4
SEQ_LEN = 2048
D_MODEL = 1024

NUM_CORES = 2
NUM_SUBCORES = 16
ROWS_PER_CHUNK = 16
NBUF = 4
VECS = ROWS_PER_CHUNK * D_MODEL // 16
LANES_PER_ROW = D_MODEL // 16

CHUNKS_C0 = 5
CHUNKS_C1 = 3
ROWS_C0 = CHUNKS_C0 * ROWS_PER_CHUNK
ROWS_C1 = CHUNKS_C1 * ROWS_PER_CHUNK
SPLIT = NUM_SUBCORES * ROWS_C0

_mesh = plsc.VectorSubcoreMesh(core_axis_name="c", subcore_axis_name="s")

_scratch = (
    [pltpu.VMEM((ROWS_PER_CHUNK, D_MODEL), jnp.float32) for _ in range(2)]
    + [pltpu.VMEM((ROWS_PER_CHUNK, D_MODEL), jnp.float32) for _ in range(NBUF)]
    + [pltpu.SemaphoreType.DMA for _ in range(2 + 2 * NBUF)]
)


@functools.partial(
    pl.kernel,
    mesh=_mesh,
    out_type=jax.ShapeDtypeStruct((BATCH, SEQ_LEN, D_MODEL), jnp.float32),
    scratch_types=_scratch,
)
def _pos_enc_sc(x_hbm, emb_hbm, out_hbm, *bufs):
    ebufs = bufs[0:2]
    xbufs = bufs[2 : 2 + NBUF]
    esems = bufs[2 + NBUF : 4 + NBUF]
    isems = bufs[4 + NBUF : 4 + 2 * NBUF]
    osems = bufs[4 + 2 * NBUF : 4 + 3 * NBUF]

    cid = lax.axis_index("c")
    sid = lax.axis_index("s")

    def pipeline(seq_base, chunks):
        jobs = chunks * BATCH

        def seq0(c):
            return seq_base + c * ROWS_PER_CHUNK

        def issue_in(j):
            c, b = divmod(j, BATCH)
            return pltpu.async_copy(
                x_hbm.at[b, pl.ds(seq0(c), ROWS_PER_CHUNK)],
                xbufs[j % NBUF],
                isems[j % NBUF],
            )

        def issue_out(j):
            c, b = divmod(j, BATCH)
            return pltpu.async_copy(
                xbufs[j % NBUF],
                out_hbm.at[b, pl.ds(seq0(c), ROWS_PER_CHUNK)],
                osems[j % NBUF],
            )

        def issue_emb(c):
            return pltpu.async_copy(
                emb_hbm.at[pl.ds(seq0(c), ROWS_PER_CHUNK)], ebufs[c % 2], esems[c % 2]
            )

        ecpys = [None] * chunks
        incpy = [None] * jobs
        outcpy = [None] * jobs

        ecpys[0] = issue_emb(0)
        if chunks > 1:
            ecpys[1] = issue_emb(1)
        for j in range(NBUF - 1):
            incpy[j] = issue_in(j)

        for j in range(jobs):
            c, b = divmod(j, BATCH)
            nj = j + NBUF - 1
            if nj < jobs:
                if nj - NBUF >= 0:
                    outcpy[nj - NBUF].wait()
                incpy[nj] = issue_in(nj)
            if b == 0:
                ecpys[c].wait()
            incpy[j].wait()

            ebuf = ebufs[c % 2]

            @plsc.parallel_loop(0, VECS, step=1, unroll=8)
            def _add(i, buf=xbufs[j % NBUF], ebuf=ebuf):
                r = i // LANES_PER_ROW
                col = (i % LANES_PER_ROW) * 16
                sl = pl.ds(col, 16)
                plsc.addupdate(buf.at[r, sl], ebuf[r, sl])

            if b == BATCH - 1 and c + 2 < chunks:
                ecpys[c + 2] = issue_emb(c + 2)

            outcpy[j] = issue_out(j)

        for j in range(max(jobs - NBUF, 0), jobs):
            outcpy[j].wait()

    @pl.when(cid == 1)
    def _():
        pipeline(sid * ROWS_C0, CHUNKS_C0)

    @pl.when(cid == 0)
    def _():
        pipeline(SPLIT + sid * ROWS_C1, CHUNKS_C1)


def kernel(x, emb):
    return _pos_enc_sc(x, emb)


# speedup vs baseline: 1.1853x; 1.1672x over previous
import functools

import jax
import jax.numpy as jnp
from jax import lax
from jax.experimental import pallas as pl
from jax.experimental.pallas import tpu as pltpu
from jax.experimental.pallas import tpu_sc as plsc

BATCH = 4
SEQ_LEN = 2048
D_MODEL = 1024

NUM_CORES = 2
NUM_SUBCORES = 16
NUM_WORKERS = NUM_CORES * NUM_SUBCORES
SEQ_PER_W = SEQ_LEN // NUM_WORKERS
ROWS_PER_CHUNK = 16
CHUNKS = SEQ_PER_W // ROWS_PER_CHUNK
NBUF = 4
VECS = ROWS_PER_CHUNK * D_MODEL // 16
LANES_PER_ROW = D_MODEL // 16
JOBS = CHUNKS * BATCH

_mesh = plsc.VectorSubcoreMesh(core_axis_name="c", subcore_axis_name="s")

_scratch = (
    [pltpu.VMEM((ROWS_PER_CHUNK, D_MODEL), jnp.float32) for _ in range(2)]
    + [pltpu.VMEM((ROWS_PER_CHUNK, D_MODEL), jnp.float32) for _ in range(NBUF)]
    + [pltpu.SemaphoreType.DMA for _ in range(2 + 2 * NBUF)]
)


@functools.partial(
    pl.kernel,
    mesh=_mesh,
    out_type=jax.ShapeDtypeStruct((BATCH, SEQ_LEN, D_MODEL), jnp.float32),
    scratch_types=_scratch,
)
def _pos_enc_sc(x_hbm, emb_hbm, out_hbm, *bufs):
    ebufs = bufs[0:2]
    xbufs = bufs[2 : 2 + NBUF]
    esems = bufs[2 + NBUF : 4 + NBUF]
    isems = bufs[4 + NBUF : 4 + 2 * NBUF]
    osems = bufs[4 + 2 * NBUF : 4 + 3 * NBUF]

    wid = lax.axis_index("s") * NUM_CORES + lax.axis_index("c")
    seq_base = wid * SEQ_PER_W

    def seq0(c):
        return seq_base + c * ROWS_PER_CHUNK

    def issue_in(j):
        c, b = divmod(j, BATCH)
        return pltpu.async_copy(
            x_hbm.at[b, pl.ds(seq0(c), ROWS_PER_CHUNK)],
            xbufs[j % NBUF],
            isems[j % NBUF],
        )

    def issue_out(j):
        c, b = divmod(j, BATCH)
        return pltpu.async_copy(
            xbufs[j % NBUF],
            out_hbm.at[b, pl.ds(seq0(c), ROWS_PER_CHUNK)],
            osems[j % NBUF],
        )

    ecpys = [None] * CHUNKS
    incpy = [None] * JOBS
    outcpy = [None] * JOBS

    ecpys[0] = pltpu.async_copy(emb_hbm.at[pl.ds(seq0(0), ROWS_PER_CHUNK)], ebufs[0], esems[0])
    ecpys[1] = pltpu.async_copy(emb_hbm.at[pl.ds(seq0(1), ROWS_PER_CHUNK)], ebufs[1], esems[1])
    for j in range(NBUF - 1):
        incpy[j] = issue_in(j)

    for j in range(JOBS):
        c, b = divmod(j, BATCH)
        nj = j + NBUF - 1
        if nj < JOBS:
            if nj - NBUF >= 0:
                outcpy[nj - NBUF].wait()
            incpy[nj] = issue_in(nj)
        if b == 0:
            ecpys[c].wait()
        incpy[j].wait()

        ebuf = ebufs[c % 2]

        @plsc.parallel_loop(0, VECS, step=1, unroll=8)
        def _add(i, buf=xbufs[j % NBUF], ebuf=ebuf):
            r = i // LANES_PER_ROW
            col = (i % LANES_PER_ROW) * 16
            sl = pl.ds(col, 16)
            plsc.addupdate(buf.at[r, sl], ebuf[r, sl])

        if b == BATCH - 1 and c + 2 < CHUNKS:
            ecpys[c + 2] = pltpu.async_copy(
                emb_hbm.at[pl.ds(seq0(c + 2), ROWS_PER_CHUNK)],
                ebufs[(c + 2) % 2],
                esems[(c + 2) % 2],
            )

        outcpy[j] = issue_out(j)

    for j in range(JOBS - NBUF, JOBS):
        outcpy[j].wait()


def kernel(x, emb):
    return _pos_enc_sc(x, emb)
